# baseline (speedup 1.0000x reference)
import jax
import jax.numpy as jnp
from jax import lax
from jax.experimental import pallas as pl
from jax.experimental.pallas import tpu as pltpu
from jax.experimental.pallas import tpu_sc as plsc

_VOCAB = 38462
_VMAIN = 38464
_VBUF = _VMAIN + 8
_NF = 26
_B = 16384
_NC, _NS, _L = 2, 16, 16
_BPH = _B // _NC
_BPT = _BPH // _NS
_ROWS = _BPH // 128


def _sc_body(idx_hbm, tab_hbm, out_hbm,
             subt0, subt1, slab0, slab1, part_v, red_v, res_v,
             sh_part, sem_t0, sem_t1, sem_s0, sem_s1, sem_r):
    s = lax.axis_index("s")
    h = lax.axis_index("c")

    f0 = s
    f1 = s + _NS
    has2 = f1 < _NF

    start0 = (f0 * _VOCAB) // 8 * 8
    slack0 = f0 * _VOCAB - start0
    cp_t0 = pltpu.async_copy(
        tab_hbm.at[pl.ds(start0, _VMAIN)], subt0.at[pl.ds(0, _VMAIN)], sem_t0)
    cp_t0b = pltpu.async_copy(
        tab_hbm.at[pl.ds(start0 + _VMAIN, 8)], subt0.at[pl.ds(_VMAIN, 8)],
        sem_t0)
    cp_s0 = pltpu.async_copy(
        idx_hbm.at[f0, pl.ds(h * _ROWS, _ROWS), pl.ds(0, 128)], slab0, sem_s0)

    @pl.when(has2)
    def _():
        start1 = (f1 * _VOCAB) // 8 * 8
        pltpu.async_copy(
            tab_hbm.at[pl.ds(start1, _VMAIN)], subt1.at[pl.ds(0, _VMAIN)],
            sem_t1).wait()
        pltpu.async_copy(
            tab_hbm.at[pl.ds(start1 + _VMAIN, 8)], subt1.at[pl.ds(_VMAIN, 8)],
            sem_t1).wait()
        pltpu.async_copy(
            idx_hbm.at[f1, pl.ds(h * _ROWS, _ROWS), pl.ds(0, 128)], slab1,
            sem_s1).wait()

    cp_t0.wait()
    cp_t0b.wait()
    cp_s0.wait()

    lane = lax.iota(jnp.int32, _L)
    zeros16 = jnp.zeros((_L,), jnp.int32)

    def acc0(ri, carry):
        rows = zeros16 + ri
        for k in range(8):
            ids = plsc.load_gather(slab0, [rows, k * _L + lane]) + slack0
            part_v[pl.ds(ri * 128 + k * _L, _L)] = (
                plsc.load_gather(subt0, [ids]))
        return carry

    lax.fori_loop(0, _ROWS, acc0, 0)

    @pl.when(has2)
    def _():
        slack1 = f1 * _VOCAB - (f1 * _VOCAB) // 8 * 8

        def acc1(ri, carry):
            rows = zeros16 + ri
            for k in range(8):
                j = ri * 128 + k * _L
                ids = plsc.load_gather(slab1, [rows, k * _L + lane]) + slack1
                part_v[pl.ds(j, _L)] = (
                    part_v[pl.ds(j, _L)] + plsc.load_gather(subt1, [ids]))
            return carry

        lax.fori_loop(0, _ROWS, acc1, 0)

    pltpu.sync_copy(part_v, sh_part.at[pl.ds(s * _BPH, _BPH)])
    plsc.subcore_barrier()
    reads = []
    for t in range(_NS):
        reads.append(pltpu.async_copy(
            sh_part.at[pl.ds(t * _BPH + s * _BPT, _BPT)], red_v.at[t], sem_r))
    for cp in reads:
        cp.wait()

    def red(c, carry):
        acc = red_v[0, pl.ds(c * _L, _L)]
        for t in range(1, _NS):
            acc = acc + red_v[t, pl.ds(c * _L, _L)]
        res_v[pl.ds(c * _L, _L)] = acc
        return carry

    lax.fori_loop(0, _BPT // _L, red, 0)
    pltpu.sync_copy(res_v, out_hbm.at[pl.ds(h * _BPH + s * _BPT, _BPT)])


_sc_call = pl.kernel(
    _sc_body,
    out_type=jax.ShapeDtypeStruct((_B,), jnp.float32),
    mesh=plsc.VectorSubcoreMesh(
        core_axis_name="c", subcore_axis_name="s",
        num_cores=_NC, num_subcores=_NS,
    ),
    scratch_types=[
        pltpu.VMEM((_VBUF,), jnp.float32),
        pltpu.VMEM((_VBUF,), jnp.float32),
        pltpu.VMEM((_ROWS, 128), jnp.int32),
        pltpu.VMEM((_ROWS, 128), jnp.int32),
        pltpu.VMEM((_BPH,), jnp.float32),
        pltpu.VMEM((_NS, _BPT), jnp.float32),
        pltpu.VMEM((_BPT,), jnp.float32),
        pltpu.VMEM_SHARED((_NS * _BPH,), jnp.float32),
        pltpu.SemaphoreType.DMA,
        pltpu.SemaphoreType.DMA,
        pltpu.SemaphoreType.DMA,
        pltpu.SemaphoreType.DMA,
        pltpu.SemaphoreType.DMA,
    ],
    compiler_params=pltpu.CompilerParams(needs_layout_passes=False),
)


def _tp_body(in_ref, out_ref):
    xt = in_ref[...].T
    for k in range(8):
        out_ref[:, k, :] = xt[:, k * 128:(k + 1) * 128]


_tp_call = pl.pallas_call(
    _tp_body,
    grid=(16,),
    in_specs=[pl.BlockSpec((1024, _NF), lambda b: (b, 0))],
    out_specs=pl.BlockSpec((_NF, 8, 128), lambda b: (0, b, 0)),
    out_shape=jax.ShapeDtypeStruct((_NF, 128, 128), jnp.int32),
)


def kernel(data, table, bias):
    dpack = _tp_call(data.astype(jnp.int32))
    tabf = table.reshape(-1)
    out = _sc_call(dpack, tabf)
    return out.reshape(_B, 1) + bias


# speedup vs baseline: 1.2588x; 1.2588x over previous
import jax
import jax.numpy as jnp
from jax import lax
from jax.experimental import pallas as pl
from jax.experimental.pallas import tpu as pltpu
from jax.experimental.pallas import tpu_sc as plsc

_VOCAB = 38462
_VMAIN = 38464
_VBUF = _VMAIN + 8
_NF = 26
_B = 16384
_NC, _NS, _L = 2, 16, 16
_BPH = _B // _NC
_BPT = _BPH // _NS
_ROWS = _BPH // 128


def _sc_body(idx_hbm, tab_hbm, out_hbm,
             subt0, subt1, slab0, slab1, part_v, red_v, res_v,
             sh_part, sem_t0, sem_t1, sem_s0, sem_s1, sem_r):
    s = lax.axis_index("s")
    h = lax.axis_index("c")

    f0 = s
    f1 = s + _NS
    has2 = f1 < _NF

    start0 = (f0 * _VOCAB) // 8 * 8
    slack0 = f0 * _VOCAB - start0
    cp_t0 = pltpu.async_copy(
        tab_hbm.at[pl.ds(start0, _VMAIN)], subt0.at[pl.ds(0, _VMAIN)], sem_t0)
    cp_t0b = pltpu.async_copy(
        tab_hbm.at[pl.ds(start0 + _VMAIN, 8)], subt0.at[pl.ds(_VMAIN, 8)],
        sem_t0)
    cp_s0 = pltpu.async_copy(
        idx_hbm.at[pl.ds(f0 * _B + h * _BPH, _BPH)], slab0, sem_s0)

    @pl.when(has2)
    def _():
        start1 = (f1 * _VOCAB) // 8 * 8
        pltpu.async_copy(
            tab_hbm.at[pl.ds(start1, _VMAIN)], subt1.at[pl.ds(0, _VMAIN)],
            sem_t1).wait()
        pltpu.async_copy(
            tab_hbm.at[pl.ds(start1 + _VMAIN, 8)], subt1.at[pl.ds(_VMAIN, 8)],
            sem_t1).wait()
        pltpu.async_copy(
            idx_hbm.at[pl.ds(f1 * _B + h * _BPH, _BPH)], slab1,
            sem_s1).wait()

    cp_t0.wait()
    cp_t0b.wait()
    cp_s0.wait()

    def acc0(c, carry):
        ids = slab0[pl.ds(c * _L, _L)] + slack0
        part_v[pl.ds(c * _L, _L)] = plsc.load_gather(subt0, [ids])
        return carry

    lax.fori_loop(0, _BPH // _L, acc0, 0)

    @pl.when(has2)
    def _():
        slack1 = f1 * _VOCAB - (f1 * _VOCAB) // 8 * 8

        def acc1(c, carry):
            ids = slab1[pl.ds(c * _L, _L)] + slack1
            part_v[pl.ds(c * _L, _L)] = (
                part_v[pl.ds(c * _L, _L)] + plsc.load_gather(subt1, [ids]))
            return carry

        lax.fori_loop(0, _BPH // _L, acc1, 0)

    pltpu.sync_copy(part_v, sh_part.at[pl.ds(s * _BPH, _BPH)])
    plsc.subcore_barrier()
    reads = []
    for t in range(_NS):
        reads.append(pltpu.async_copy(
            sh_part.at[pl.ds(t * _BPH + s * _BPT, _BPT)], red_v.at[t], sem_r))
    for cp in reads:
        cp.wait()

    def red(c, carry):
        acc = red_v[0, pl.ds(c * _L, _L)]
        for t in range(1, _NS):
            acc = acc + red_v[t, pl.ds(c * _L, _L)]
        res_v[pl.ds(c * _L, _L)] = acc
        return carry

    lax.fori_loop(0, _BPT // _L, red, 0)
    pltpu.sync_copy(res_v, out_hbm.at[pl.ds(h * _BPH + s * _BPT, _BPT)])


_sc_call = pl.kernel(
    _sc_body,
    out_type=jax.ShapeDtypeStruct((_B,), jnp.float32),
    mesh=plsc.VectorSubcoreMesh(
        core_axis_name="c", subcore_axis_name="s",
        num_cores=_NC, num_subcores=_NS,
    ),
    scratch_types=[
        pltpu.VMEM((_VBUF,), jnp.float32),
        pltpu.VMEM((_VBUF,), jnp.float32),
        pltpu.VMEM((_BPH,), jnp.int32),
        pltpu.VMEM((_BPH,), jnp.int32),
        pltpu.VMEM((_BPH,), jnp.float32),
        pltpu.VMEM((_NS, _BPT), jnp.float32),
        pltpu.VMEM((_BPT,), jnp.float32),
        pltpu.VMEM_SHARED((_NS * _BPH,), jnp.float32),
        pltpu.SemaphoreType.DMA,
        pltpu.SemaphoreType.DMA,
        pltpu.SemaphoreType.DMA,
        pltpu.SemaphoreType.DMA,
        pltpu.SemaphoreType.DMA,
    ],
    compiler_params=pltpu.CompilerParams(needs_layout_passes=False),
)


def kernel(data, table, bias):
    dpack = data.astype(jnp.int32).T.reshape(_NF, 128, 128)
    dflat = dpack.reshape(_NF * _B)
    tabf = table.reshape(-1)
    out = _sc_call(dflat, tabf)
    return out.reshape(_B, 1) + bias
